# baseline (speedup 1.0000x reference)
import functools

import jax
import jax.numpy as jnp
from jax import lax
from jax.experimental import pallas as pl
from jax.experimental.pallas import tpu as pltpu
from jax.experimental.pallas import tpu_sc as plsc

N = 10000
E = 320000
D = 128

NC = 2
NS = 16
NW = NC * NS

BATCH = 128
NB = 81
NPAIR = NB // 2
EPT = NB * BATCH
EPAD = NW * EPT

NA = 10240
RPT = NA // NS
DEGW = 16

_MESH = dict(core_axis_name="c", subcore_axis_name="s", num_cores=NC,
             num_subcores=NS)


def _sc_degree(dstp, ones16, zeros16):
  @functools.partial(
      pl.kernel,
      out_type=jax.ShapeDtypeStruct((NC * NA, DEGW), jnp.float32),
      mesh=plsc.VectorSubcoreMesh(**_MESH),
      scratch_types=[
          pltpu.VMEM_SHARED((NA, DEGW), jnp.float32),
          pltpu.VMEM((BATCH,), jnp.int32),
          pltpu.VMEM((BATCH, DEGW), jnp.float32),
          pltpu.VMEM((BATCH, DEGW), jnp.float32),
      ],
  )
  def k(dst_hbm, ones_hbm, z_hbm, out_hbm, acc, didx, ones_v, z_v):
    cid = lax.axis_index("c")
    sid = lax.axis_index("s")
    wid = cid * NS + sid
    pltpu.sync_copy(ones_hbm, ones_v)
    pltpu.sync_copy(z_hbm, z_v)
    for j in range(RPT // BATCH):
      pltpu.sync_copy(z_v, acc.at[pl.ds(sid * RPT + j * BATCH, BATCH)])
    plsc.subcore_barrier()

    def step(t, carry):
      pltpu.sync_copy(dst_hbm.at[pl.ds(wid * EPT + t * BATCH, BATCH)], didx)
      pltpu.sync_copy(ones_v, acc.at[didx], add=True)
      return carry

    lax.fori_loop(0, NB, step, 0)
    plsc.subcore_barrier()
    for j in range(RPT // BATCH):
      r0 = sid * RPT + j * BATCH
      pltpu.sync_copy(acc.at[pl.ds(r0, BATCH)],
                      out_hbm.at[pl.ds(cid * NA + r0, BATCH)])

  return k(dstp, ones16, zeros16).reshape(NC, NA, DEGW)


def _sc_propagate(g, srcp, dstp, zeros128):
  @functools.partial(
      pl.kernel,
      out_type=jax.ShapeDtypeStruct((NC * NA, D), jnp.float32),
      mesh=plsc.VectorSubcoreMesh(**_MESH),
      scratch_types=[
          pltpu.VMEM_SHARED((NA, D), jnp.float32),
          pltpu.VMEM((BATCH,), jnp.int32),
          pltpu.VMEM((BATCH,), jnp.int32),
          pltpu.VMEM((BATCH,), jnp.int32),
          pltpu.VMEM((BATCH,), jnp.int32),
          pltpu.VMEM((BATCH, D), jnp.float32),
          pltpu.VMEM((BATCH, D), jnp.float32),
          pltpu.SemaphoreType.DMA,
          pltpu.SemaphoreType.DMA,
      ],
  )
  def k(g_hbm, src_hbm, dst_hbm, z_hbm, out_hbm, acc, sidx0, sidx1, didx0,
        didx1, rows0, rows1, sem0, sem1):
    cid = lax.axis_index("c")
    sid = lax.axis_index("s")
    wid = cid * NS + sid
    base = wid * EPT
    pltpu.sync_copy(z_hbm, rows0)
    for j in range(RPT // BATCH):
      pltpu.sync_copy(rows0, acc.at[pl.ds(sid * RPT + j * BATCH, BATCH)])
    plsc.subcore_barrier()

    pltpu.sync_copy(src_hbm.at[pl.ds(base, BATCH)], sidx0)
    pltpu.async_copy(g_hbm.at[sidx0], rows0, sem0)

    def pair(i, carry):
      o0 = base + 2 * i * BATCH
      pltpu.sync_copy(src_hbm.at[pl.ds(o0 + BATCH, BATCH)], sidx1)
      pltpu.async_copy(g_hbm.at[sidx1], rows1, sem1)
      pltpu.sync_copy(dst_hbm.at[pl.ds(o0, BATCH)], didx0)
      pltpu.make_async_copy(g_hbm.at[sidx0], rows0, sem0).wait()
      pltpu.sync_copy(rows0, acc.at[didx0], add=True)
      pltpu.sync_copy(src_hbm.at[pl.ds(o0 + 2 * BATCH, BATCH)], sidx0)
      pltpu.async_copy(g_hbm.at[sidx0], rows0, sem0)
      pltpu.sync_copy(dst_hbm.at[pl.ds(o0 + BATCH, BATCH)], didx1)
      pltpu.make_async_copy(g_hbm.at[sidx1], rows1, sem1).wait()
      pltpu.sync_copy(rows1, acc.at[didx1], add=True)
      return carry

    lax.fori_loop(0, NPAIR, pair, 0)
    pltpu.sync_copy(dst_hbm.at[pl.ds(base + (NB - 1) * BATCH, BATCH)], didx0)
    pltpu.make_async_copy(g_hbm.at[sidx0], rows0, sem0).wait()
    pltpu.sync_copy(rows0, acc.at[didx0], add=True)
    plsc.subcore_barrier()
    for j in range(RPT // BATCH):
      r0 = sid * RPT + j * BATCH
      pltpu.sync_copy(acc.at[pl.ds(r0, BATCH)],
                      out_hbm.at[pl.ds(cid * NA + r0, BATCH)])

  return k(g, srcp, dstp, zeros128).reshape(NC, NA, D)


_RB = 400


def _deg_col(dref):
  deg = dref[0] + dref[1]
  return deg[:, 0:1]


def _tc_scale(x, degp):
  def body(x_ref, d_ref, o_ref):
    o_ref[...] = x_ref[...] * lax.rsqrt(_deg_col(d_ref))

  return pl.pallas_call(
      body,
      grid=(N // _RB,),
      in_specs=[
          pl.BlockSpec((_RB, D), lambda i: (i, 0)),
          pl.BlockSpec((NC, _RB, DEGW), lambda i: (0, i, 0)),
      ],
      out_specs=pl.BlockSpec((_RB, D), lambda i: (i, 0)),
      out_shape=jax.ShapeDtypeStruct((N, D), jnp.float32),
  )(x, degp)


def _tc_combine(p, degp):
  def body(p_ref, d_ref, o_ref):
    o_ref[...] = (p_ref[0] + p_ref[1]) / _deg_col(d_ref)

  return pl.pallas_call(
      body,
      grid=(N // _RB,),
      in_specs=[
          pl.BlockSpec((NC, _RB, D), lambda i: (0, i, 0)),
          pl.BlockSpec((NC, _RB, DEGW), lambda i: (0, i, 0)),
      ],
      out_specs=pl.BlockSpec((_RB, D), lambda i: (i, 0)),
      out_shape=jax.ShapeDtypeStruct((N, D), jnp.float32),
  )(p, degp)


def _tc_final(q, degp, W, b2):
  def body(q_ref, d_ref, w_ref, b_ref, o_ref):
    h2 = (q_ref[0] + q_ref[1]) * lax.rsqrt(_deg_col(d_ref))
    o_ref[...] = lax.dot_general(
        h2, w_ref[...], (((1,), (1,)), ((), ())),
        preferred_element_type=jnp.float32) + b_ref[...]

  return pl.pallas_call(
      body,
      grid=(N // _RB,),
      in_specs=[
          pl.BlockSpec((NC, _RB, D), lambda i: (0, i, 0)),
          pl.BlockSpec((NC, _RB, DEGW), lambda i: (0, i, 0)),
          pl.BlockSpec((D, D), lambda i: (0, 0)),
          pl.BlockSpec((1, D), lambda i: (0, 0)),
      ],
      out_specs=pl.BlockSpec((_RB, D), lambda i: (i, 0)),
      out_shape=jax.ShapeDtypeStruct((N, D), jnp.float32),
  )(q, degp, W, b2)


def kernel(x, edge_index, W, b):
  src = edge_index[0]
  dst = edge_index[1]
  loop = jnp.arange(N, dtype=jnp.int32)
  pad = EPAD - (E + N)
  half = E // 2
  srcp = jnp.concatenate([src[:half], loop, src[half:],
                          jnp.zeros((pad,), jnp.int32)])
  dstp = jnp.concatenate([dst[:half], loop, dst[half:],
                          jnp.full((pad,), N, jnp.int32)])

  zeros128 = jnp.zeros((BATCH, D), jnp.float32)
  zeros16 = jnp.zeros((BATCH, DEGW), jnp.float32)
  ones16 = jnp.ones((BATCH, DEGW), jnp.float32)

  degp = _sc_degree(dstp, ones16, zeros16)
  g0 = _tc_scale(x, degp)
  p = _sc_propagate(g0, srcp, dstp, zeros128)
  g1 = _tc_combine(p, degp)
  q = _sc_propagate(g1, srcp, dstp, zeros128)
  return _tc_final(q, degp, W, b.reshape(1, D))


# speedup vs baseline: 1.0047x; 1.0047x over previous
import functools

import jax
import jax.numpy as jnp
from jax import lax
from jax.experimental import pallas as pl
from jax.experimental.pallas import tpu as pltpu
from jax.experimental.pallas import tpu_sc as plsc

N = 10000
E = 320000
D = 128

NC = 2
NS = 16
NW = NC * NS

BATCH = 128
NB = 81
NPAIR = NB // 2
EPT = NB * BATCH
EPAD = NW * EPT

NA = 10112
RPT = NA // NS
DEGW = 16

_MESH = dict(core_axis_name="c", subcore_axis_name="s", num_cores=NC,
             num_subcores=NS)

_CHUNKS = [(j * BATCH, min(BATCH, RPT - j * BATCH))
           for j in range((RPT + BATCH - 1) // BATCH)]


def _sc_degree(dstp, ones16, zeros16):
  @functools.partial(
      pl.kernel,
      out_type=jax.ShapeDtypeStruct((NC * NA, DEGW), jnp.float32),
      mesh=plsc.VectorSubcoreMesh(**_MESH),
      scratch_types=[
          pltpu.VMEM_SHARED((NA, DEGW), jnp.float32),
          pltpu.VMEM((BATCH,), jnp.int32),
          pltpu.VMEM((BATCH, DEGW), jnp.float32),
          pltpu.VMEM((BATCH, DEGW), jnp.float32),
      ],
  )
  def k(dst_hbm, ones_hbm, z_hbm, out_hbm, acc, didx, ones_v, z_v):
    cid = lax.axis_index("c")
    sid = lax.axis_index("s")
    wid = cid * NS + sid
    pltpu.sync_copy(ones_hbm, ones_v)
    pltpu.sync_copy(z_hbm, z_v)
    for o, n in _CHUNKS:
      pltpu.sync_copy(z_v.at[pl.ds(0, n)], acc.at[pl.ds(sid * RPT + o, n)])
    plsc.subcore_barrier()

    def step(t, carry):
      pltpu.sync_copy(dst_hbm.at[pl.ds(wid * EPT + t * BATCH, BATCH)], didx)
      pltpu.sync_copy(ones_v, acc.at[didx], add=True)
      return carry

    lax.fori_loop(0, NB, step, 0)
    plsc.subcore_barrier()
    for o, n in _CHUNKS:
      r0 = sid * RPT + o
      pltpu.sync_copy(acc.at[pl.ds(r0, n)],
                      out_hbm.at[pl.ds(cid * NA + r0, n)])

  return k(dstp, ones16, zeros16).reshape(NC, NA, DEGW)


def _sc_propagate(g, srcp, dstp, zeros128):
  @functools.partial(
      pl.kernel,
      out_type=jax.ShapeDtypeStruct((NC * NA, D), jnp.float32),
      mesh=plsc.VectorSubcoreMesh(**_MESH),
      scratch_types=[
          pltpu.VMEM_SHARED((NA, D), jnp.float32),
          pltpu.VMEM((BATCH,), jnp.int32),
          pltpu.VMEM((BATCH,), jnp.int32),
          pltpu.VMEM((BATCH,), jnp.int32),
          pltpu.VMEM((BATCH,), jnp.int32),
          pltpu.VMEM((BATCH,), jnp.int32),
          pltpu.VMEM((BATCH,), jnp.int32),
          pltpu.VMEM((BATCH, D), jnp.float32),
          pltpu.VMEM((BATCH, D), jnp.float32),
          pltpu.VMEM((BATCH, D), jnp.float32),
          pltpu.SemaphoreType.DMA,
          pltpu.SemaphoreType.DMA,
          pltpu.SemaphoreType.DMA,
      ],
  )
  def k(g_hbm, src_hbm, dst_hbm, z_hbm, out_hbm, acc, sidx0, sidx1, sidx2,
        didx0, didx1, didx2, rows0, rows1, rows2, sem0, sem1, sem2):
    cid = lax.axis_index("c")
    sid = lax.axis_index("s")
    wid = cid * NS + sid
    base = wid * EPT
    pltpu.sync_copy(z_hbm, rows0)
    for o, n in _CHUNKS:
      pltpu.sync_copy(rows0.at[pl.ds(0, n)], acc.at[pl.ds(sid * RPT + o, n)])
    plsc.subcore_barrier()

    def issue(t_off, sidx, rows, sem):
      pltpu.sync_copy(src_hbm.at[pl.ds(t_off, BATCH)], sidx)
      pltpu.async_copy(g_hbm.at[sidx], rows, sem)

    def proc(t_off, didx, rows, sem, sidx):
      pltpu.sync_copy(dst_hbm.at[pl.ds(t_off, BATCH)], didx)
      pltpu.make_async_copy(g_hbm.at[sidx], rows, sem).wait()
      pltpu.sync_copy(rows, acc.at[didx], add=True)

    issue(base, sidx0, rows0, sem0)
    issue(base + BATCH, sidx1, rows1, sem1)
    NTRI = NB // 3 - 1

    def tri(i, carry):
      o = base + 3 * i * BATCH
      proc(o, didx0, rows0, sem0, sidx0)
      issue(o + 2 * BATCH, sidx2, rows2, sem2)
      proc(o + BATCH, didx1, rows1, sem1, sidx1)
      issue(o + 3 * BATCH, sidx0, rows0, sem0)
      proc(o + 2 * BATCH, didx2, rows2, sem2, sidx2)
      issue(o + 4 * BATCH, sidx1, rows1, sem1)
      return carry

    lax.fori_loop(0, NTRI, tri, 0)
    o = base + 3 * NTRI * BATCH
    proc(o, didx0, rows0, sem0, sidx0)
    issue(o + 2 * BATCH, sidx2, rows2, sem2)
    proc(o + BATCH, didx1, rows1, sem1, sidx1)
    proc(o + 2 * BATCH, didx2, rows2, sem2, sidx2)
    plsc.subcore_barrier()
    for o, n in _CHUNKS:
      r0 = sid * RPT + o
      pltpu.sync_copy(acc.at[pl.ds(r0, n)],
                      out_hbm.at[pl.ds(cid * NA + r0, n)])

  return k(g, srcp, dstp, zeros128).reshape(NC, NA, D)


_RB = 400


def _deg_col(dref):
  deg = dref[0] + dref[1]
  return deg[:, 0:1]


def _tc_scale(x, degp):
  def body(x_ref, d_ref, o_ref):
    o_ref[...] = x_ref[...] * lax.rsqrt(_deg_col(d_ref))

  return pl.pallas_call(
      body,
      grid=(N // _RB,),
      in_specs=[
          pl.BlockSpec((_RB, D), lambda i: (i, 0)),
          pl.BlockSpec((NC, _RB, DEGW), lambda i: (0, i, 0)),
      ],
      out_specs=pl.BlockSpec((_RB, D), lambda i: (i, 0)),
      out_shape=jax.ShapeDtypeStruct((N, D), jnp.float32),
  )(x, degp)


def _tc_combine(p, degp):
  def body(p_ref, d_ref, o_ref):
    o_ref[...] = (p_ref[0] + p_ref[1]) / _deg_col(d_ref)

  return pl.pallas_call(
      body,
      grid=(N // _RB,),
      in_specs=[
          pl.BlockSpec((NC, _RB, D), lambda i: (0, i, 0)),
          pl.BlockSpec((NC, _RB, DEGW), lambda i: (0, i, 0)),
      ],
      out_specs=pl.BlockSpec((_RB, D), lambda i: (i, 0)),
      out_shape=jax.ShapeDtypeStruct((N, D), jnp.float32),
  )(p, degp)


def _tc_final(q, degp, W, b2):
  def body(q_ref, d_ref, w_ref, b_ref, o_ref):
    h2 = (q_ref[0] + q_ref[1]) * lax.rsqrt(_deg_col(d_ref))
    o_ref[...] = lax.dot_general(
        h2, w_ref[...], (((1,), (1,)), ((), ())),
        preferred_element_type=jnp.float32) + b_ref[...]

  return pl.pallas_call(
      body,
      grid=(N // _RB,),
      in_specs=[
          pl.BlockSpec((NC, _RB, D), lambda i: (0, i, 0)),
          pl.BlockSpec((NC, _RB, DEGW), lambda i: (0, i, 0)),
          pl.BlockSpec((D, D), lambda i: (0, 0)),
          pl.BlockSpec((1, D), lambda i: (0, 0)),
      ],
      out_specs=pl.BlockSpec((_RB, D), lambda i: (i, 0)),
      out_shape=jax.ShapeDtypeStruct((N, D), jnp.float32),
  )(q, degp, W, b2)


def kernel(x, edge_index, W, b):
  src = edge_index[0]
  dst = edge_index[1]
  loop = jnp.arange(N, dtype=jnp.int32)
  pad = EPAD - (E + N)
  srcp = jnp.concatenate([src, loop, jnp.zeros((pad,), jnp.int32)])
  dstp = jnp.concatenate([dst, loop, jnp.full((pad,), N, jnp.int32)])

  zeros128 = jnp.zeros((BATCH, D), jnp.float32)
  zeros16 = jnp.zeros((BATCH, DEGW), jnp.float32)
  ones16 = jnp.ones((BATCH, DEGW), jnp.float32)

  degp = _sc_degree(dstp, ones16, zeros16)
  g0 = _tc_scale(x, degp)
  p = _sc_propagate(g0, srcp, dstp, zeros128)
  g1 = _tc_combine(p, degp)
  q = _sc_propagate(g1, srcp, dstp, zeros128)
  return _tc_final(q, degp, W, b.reshape(1, D))


# speedup vs baseline: 1.0286x; 1.0238x over previous
import functools

import jax
import jax.numpy as jnp
from jax import lax
from jax.experimental import pallas as pl
from jax.experimental.pallas import tpu as pltpu
from jax.experimental.pallas import tpu_sc as plsc

N = 10000
E = 320000
D = 128

NC = 2
NS = 16
NW = NC * NS

BATCH = 128
NB = 81
NPAIR = NB // 2
EPT = NB * BATCH
EPAD = NW * EPT

NA = 10240
RPT = NA // NS
DEGW = 16

_MESH = dict(core_axis_name="c", subcore_axis_name="s", num_cores=NC,
             num_subcores=NS)

_CHUNKS = [(j * BATCH, min(BATCH, RPT - j * BATCH))
           for j in range((RPT + BATCH - 1) // BATCH)]


def _sc_degree(dstp, ones16, zeros16):
  @functools.partial(
      pl.kernel,
      out_type=jax.ShapeDtypeStruct((NC * NA, DEGW), jnp.float32),
      mesh=plsc.VectorSubcoreMesh(**_MESH),
      scratch_types=[
          pltpu.VMEM_SHARED((NA, DEGW), jnp.float32),
          pltpu.VMEM((BATCH,), jnp.int32),
          pltpu.VMEM((BATCH, DEGW), jnp.float32),
          pltpu.VMEM((BATCH, DEGW), jnp.float32),
      ],
  )
  def k(dst_hbm, ones_hbm, z_hbm, out_hbm, acc, didx, ones_v, z_v):
    cid = lax.axis_index("c")
    sid = lax.axis_index("s")
    wid = cid * NS + sid
    pltpu.sync_copy(ones_hbm, ones_v)
    pltpu.sync_copy(z_hbm, z_v)
    for o, n in _CHUNKS:
      pltpu.sync_copy(z_v.at[pl.ds(0, n)], acc.at[pl.ds(sid * RPT + o, n)])
    plsc.subcore_barrier()

    def step(t, carry):
      pltpu.sync_copy(dst_hbm.at[pl.ds(wid * EPT + t * BATCH, BATCH)], didx)
      pltpu.sync_copy(ones_v, acc.at[didx], add=True)
      return carry

    lax.fori_loop(0, NB, step, 0)
    plsc.subcore_barrier()
    for o, n in _CHUNKS:
      r0 = sid * RPT + o
      pltpu.sync_copy(acc.at[pl.ds(r0, n)],
                      out_hbm.at[pl.ds(cid * NA + r0, n)])

  return k(dstp, ones16, zeros16).reshape(NC, NA, DEGW)


def _sc_propagate(g, srcp, dstp, zeros128):
  @functools.partial(
      pl.kernel,
      out_type=jax.ShapeDtypeStruct((NC * NA, D), jnp.float32),
      mesh=plsc.VectorSubcoreMesh(**_MESH),
      scratch_types=[
          pltpu.VMEM_SHARED((NA, D), jnp.float32),
          pltpu.VMEM((BATCH,), jnp.int32),
          pltpu.VMEM((BATCH,), jnp.int32),
          pltpu.VMEM((BATCH,), jnp.int32),
          pltpu.VMEM((BATCH,), jnp.int32),
          pltpu.VMEM((BATCH, D), jnp.float32),
          pltpu.VMEM((BATCH, D), jnp.float32),
          pltpu.SemaphoreType.DMA,
          pltpu.SemaphoreType.DMA,
      ],
  )
  def k(g_hbm, src_hbm, dst_hbm, z_hbm, out_hbm, acc, sidx0, sidx1,
        didx0, didx1, rows0, rows1, sem0, sem1):
    cid = lax.axis_index("c")
    sid = lax.axis_index("s")
    wid = cid * NS + sid
    base = wid * EPT
    pltpu.sync_copy(z_hbm, rows0)
    for o, n in _CHUNKS:
      pltpu.sync_copy(rows0.at[pl.ds(0, n)], acc.at[pl.ds(sid * RPT + o, n)])
    plsc.subcore_barrier()

    def issue(t_off, sidx, rows, sem):
      pltpu.sync_copy(src_hbm.at[pl.ds(t_off, BATCH)], sidx)
      pltpu.async_copy(g_hbm.at[sidx], rows, sem)

    def proc(t_off, didx, rows, sem, sidx):
      pltpu.sync_copy(dst_hbm.at[pl.ds(t_off, BATCH)], didx)
      pltpu.make_async_copy(g_hbm.at[sidx], rows, sem).wait()
      pltpu.sync_copy(rows, acc.at[didx], add=True)

    issue(base, sidx0, rows0, sem0)

    def pair(i, carry):
      o = base + 2 * i * BATCH
      issue(o + BATCH, sidx1, rows1, sem1)
      proc(o, didx0, rows0, sem0, sidx0)
      issue(o + 2 * BATCH, sidx0, rows0, sem0)
      proc(o + BATCH, didx1, rows1, sem1, sidx1)
      return carry

    lax.fori_loop(0, NPAIR, pair, 0)
    proc(base + (NB - 1) * BATCH, didx0, rows0, sem0, sidx0)
    plsc.subcore_barrier()
    for o, n in _CHUNKS:
      r0 = sid * RPT + o
      pltpu.sync_copy(acc.at[pl.ds(r0, n)],
                      out_hbm.at[pl.ds(cid * NA + r0, n)])

  return k(g, srcp, dstp, zeros128).reshape(NC, NA, D)


_RB = 400


def _deg_col(dref):
  deg = dref[0] + dref[1]
  return deg[:, 0:1]


def _tc_scale(x, degp):
  def body(x_ref, d_ref, o_ref):
    o_ref[...] = x_ref[...] * lax.rsqrt(_deg_col(d_ref))

  return pl.pallas_call(
      body,
      grid=(N // _RB,),
      in_specs=[
          pl.BlockSpec((_RB, D), lambda i: (i, 0)),
          pl.BlockSpec((NC, _RB, DEGW), lambda i: (0, i, 0)),
      ],
      out_specs=pl.BlockSpec((_RB, D), lambda i: (i, 0)),
      out_shape=jax.ShapeDtypeStruct((N, D), jnp.float32),
  )(x, degp)


def _tc_combine(p, degp):
  def body(p_ref, d_ref, o_ref):
    o_ref[...] = (p_ref[0] + p_ref[1]) / _deg_col(d_ref)

  return pl.pallas_call(
      body,
      grid=(N // _RB,),
      in_specs=[
          pl.BlockSpec((NC, _RB, D), lambda i: (0, i, 0)),
          pl.BlockSpec((NC, _RB, DEGW), lambda i: (0, i, 0)),
      ],
      out_specs=pl.BlockSpec((_RB, D), lambda i: (i, 0)),
      out_shape=jax.ShapeDtypeStruct((N, D), jnp.float32),
  )(p, degp)


def _tc_final(q, degp, W, b2):
  def body(q_ref, d_ref, w_ref, b_ref, o_ref):
    h2 = (q_ref[0] + q_ref[1]) * lax.rsqrt(_deg_col(d_ref))
    o_ref[...] = lax.dot_general(
        h2, w_ref[...], (((1,), (1,)), ((), ())),
        preferred_element_type=jnp.float32) + b_ref[...]

  return pl.pallas_call(
      body,
      grid=(N // _RB,),
      in_specs=[
          pl.BlockSpec((NC, _RB, D), lambda i: (0, i, 0)),
          pl.BlockSpec((NC, _RB, DEGW), lambda i: (0, i, 0)),
          pl.BlockSpec((D, D), lambda i: (0, 0)),
          pl.BlockSpec((1, D), lambda i: (0, 0)),
      ],
      out_specs=pl.BlockSpec((_RB, D), lambda i: (i, 0)),
      out_shape=jax.ShapeDtypeStruct((N, D), jnp.float32),
  )(q, degp, W, b2)


def kernel(x, edge_index, W, b):
  src = edge_index[0]
  dst = edge_index[1]
  loop = jnp.arange(N, dtype=jnp.int32)
  pad = EPAD - (E + N)
  srcp = jnp.concatenate([src, loop, jnp.zeros((pad,), jnp.int32)])
  dstp = jnp.concatenate([dst, loop, jnp.full((pad,), N, jnp.int32)])

  zeros128 = jnp.zeros((BATCH, D), jnp.float32)
  zeros16 = jnp.zeros((BATCH, DEGW), jnp.float32)
  ones16 = jnp.ones((BATCH, DEGW), jnp.float32)

  degp = _sc_degree(dstp, ones16, zeros16)
  g0 = _tc_scale(x, degp)
  p = _sc_propagate(g0, srcp, dstp, zeros128)
  g1 = _tc_combine(p, degp)
  q = _sc_propagate(g1, srcp, dstp, zeros128)
  return _tc_final(q, degp, W, b.reshape(1, D))


# speedup vs baseline: 1.0455x; 1.0164x over previous
import functools

import jax
import jax.numpy as jnp
from jax import lax
from jax.experimental import pallas as pl
from jax.experimental.pallas import tpu as pltpu
from jax.experimental.pallas import tpu_sc as plsc

N = 10000
E = 320000
D = 128

NC = 2
NS = 16
NW = NC * NS

BATCH = 128
NB = 81
NPAIR = NB // 2
EPT = NB * BATCH
EPAD = NW * EPT

NA = 10240
RPT = NA // NS
DEGW = 16

_MESH = dict(core_axis_name="c", subcore_axis_name="s", num_cores=NC,
             num_subcores=NS)

_CHUNKS = [(j * BATCH, min(BATCH, RPT - j * BATCH))
           for j in range((RPT + BATCH - 1) // BATCH)]


def _sc_degree(dstp, ones16, zeros16):
  @functools.partial(
      pl.kernel,
      out_type=jax.ShapeDtypeStruct((NC * NA, DEGW), jnp.float32),
      mesh=plsc.VectorSubcoreMesh(**_MESH),
      scratch_types=[
          pltpu.VMEM_SHARED((NA, DEGW), jnp.float32),
          pltpu.VMEM((BATCH,), jnp.int32),
          pltpu.VMEM((BATCH,), jnp.int32),
          pltpu.VMEM((BATCH, DEGW), jnp.float32),
          pltpu.VMEM((BATCH, DEGW), jnp.float32),
          pltpu.SemaphoreType.DMA,
          pltpu.SemaphoreType.DMA,
      ],
  )
  def k(dst_hbm, ones_hbm, z_hbm, out_hbm, acc, didx0, didx1, ones_v, z_v,
        sem0, sem1):
    cid = lax.axis_index("c")
    sid = lax.axis_index("s")
    base = (cid * NS + sid) * EPT
    pltpu.sync_copy(ones_hbm, ones_v)
    pltpu.sync_copy(z_hbm, z_v)
    for o, n in _CHUNKS:
      pltpu.sync_copy(z_v.at[pl.ds(0, n)], acc.at[pl.ds(sid * RPT + o, n)])
    plsc.subcore_barrier()

    def issue(t_off, didx, sem):
      pltpu.async_copy(dst_hbm.at[pl.ds(t_off, BATCH)], didx, sem)

    def proc(t_off, didx, sem):
      pltpu.make_async_copy(dst_hbm.at[pl.ds(t_off, BATCH)], didx, sem).wait()
      pltpu.sync_copy(ones_v, acc.at[didx], add=True)

    issue(base, didx0, sem0)

    def pair(i, carry):
      o = base + 2 * i * BATCH
      issue(o + BATCH, didx1, sem1)
      proc(o, didx0, sem0)
      issue(o + 2 * BATCH, didx0, sem0)
      proc(o + BATCH, didx1, sem1)
      return carry

    lax.fori_loop(0, NPAIR, pair, 0)
    proc(base + (NB - 1) * BATCH, didx0, sem0)
    plsc.subcore_barrier()
    for o, n in _CHUNKS:
      r0 = sid * RPT + o
      pltpu.sync_copy(acc.at[pl.ds(r0, n)],
                      out_hbm.at[pl.ds(cid * NA + r0, n)])

  return k(dstp, ones16, zeros16).reshape(NC, NA, DEGW)


def _sc_propagate(g, srcp, dstp, zeros128):
  @functools.partial(
      pl.kernel,
      out_type=jax.ShapeDtypeStruct((NC * NA, D), jnp.float32),
      mesh=plsc.VectorSubcoreMesh(**_MESH),
      scratch_types=[
          pltpu.VMEM_SHARED((NA, D), jnp.float32),
          pltpu.VMEM((BATCH,), jnp.int32),
          pltpu.VMEM((BATCH,), jnp.int32),
          pltpu.VMEM((BATCH,), jnp.int32),
          pltpu.VMEM((BATCH,), jnp.int32),
          pltpu.VMEM((BATCH, D), jnp.float32),
          pltpu.VMEM((BATCH, D), jnp.float32),
          pltpu.SemaphoreType.DMA,
          pltpu.SemaphoreType.DMA,
      ],
  )
  def k(g_hbm, src_hbm, dst_hbm, z_hbm, out_hbm, acc, sidx0, sidx1,
        didx0, didx1, rows0, rows1, sem0, sem1):
    cid = lax.axis_index("c")
    sid = lax.axis_index("s")
    wid = cid * NS + sid
    base = wid * EPT
    pltpu.sync_copy(z_hbm, rows0)
    for o, n in _CHUNKS:
      pltpu.sync_copy(rows0.at[pl.ds(0, n)], acc.at[pl.ds(sid * RPT + o, n)])
    plsc.subcore_barrier()

    def issue(t_off, sidx, rows, sem):
      pltpu.sync_copy(src_hbm.at[pl.ds(t_off, BATCH)], sidx)
      pltpu.async_copy(g_hbm.at[sidx], rows, sem)

    def proc(t_off, didx, rows, sem, sidx):
      pltpu.sync_copy(dst_hbm.at[pl.ds(t_off, BATCH)], didx)
      pltpu.make_async_copy(g_hbm.at[sidx], rows, sem).wait()
      pltpu.sync_copy(rows, acc.at[didx], add=True)

    issue(base, sidx0, rows0, sem0)

    def pair(i, carry):
      o = base + 2 * i * BATCH
      issue(o + BATCH, sidx1, rows1, sem1)
      proc(o, didx0, rows0, sem0, sidx0)
      issue(o + 2 * BATCH, sidx0, rows0, sem0)
      proc(o + BATCH, didx1, rows1, sem1, sidx1)
      return carry

    lax.fori_loop(0, NPAIR, pair, 0)
    proc(base + (NB - 1) * BATCH, didx0, rows0, sem0, sidx0)
    plsc.subcore_barrier()
    for o, n in _CHUNKS:
      r0 = sid * RPT + o
      pltpu.sync_copy(acc.at[pl.ds(r0, n)],
                      out_hbm.at[pl.ds(cid * NA + r0, n)])

  return k(g, srcp, dstp, zeros128).reshape(NC, NA, D)


_RB = 400


def _deg_col(dref):
  deg = dref[0] + dref[1]
  return deg[:, 0:1]


def _tc_scale(x, degp):
  def body(x_ref, d_ref, o_ref):
    o_ref[...] = x_ref[...] * lax.rsqrt(_deg_col(d_ref))

  return pl.pallas_call(
      body,
      grid=(N // _RB,),
      in_specs=[
          pl.BlockSpec((_RB, D), lambda i: (i, 0)),
          pl.BlockSpec((NC, _RB, DEGW), lambda i: (0, i, 0)),
      ],
      out_specs=pl.BlockSpec((_RB, D), lambda i: (i, 0)),
      out_shape=jax.ShapeDtypeStruct((N, D), jnp.float32),
  )(x, degp)


def _tc_combine(p, degp):
  def body(p_ref, d_ref, o_ref):
    o_ref[...] = (p_ref[0] + p_ref[1]) / _deg_col(d_ref)

  return pl.pallas_call(
      body,
      grid=(N // _RB,),
      in_specs=[
          pl.BlockSpec((NC, _RB, D), lambda i: (0, i, 0)),
          pl.BlockSpec((NC, _RB, DEGW), lambda i: (0, i, 0)),
      ],
      out_specs=pl.BlockSpec((_RB, D), lambda i: (i, 0)),
      out_shape=jax.ShapeDtypeStruct((N, D), jnp.float32),
  )(p, degp)


def _tc_final(q, degp, W, b2):
  def body(q_ref, d_ref, w_ref, b_ref, o_ref):
    h2 = (q_ref[0] + q_ref[1]) * lax.rsqrt(_deg_col(d_ref))
    o_ref[...] = lax.dot_general(
        h2, w_ref[...], (((1,), (1,)), ((), ())),
        preferred_element_type=jnp.float32) + b_ref[...]

  return pl.pallas_call(
      body,
      grid=(N // _RB,),
      in_specs=[
          pl.BlockSpec((NC, _RB, D), lambda i: (0, i, 0)),
          pl.BlockSpec((NC, _RB, DEGW), lambda i: (0, i, 0)),
          pl.BlockSpec((D, D), lambda i: (0, 0)),
          pl.BlockSpec((1, D), lambda i: (0, 0)),
      ],
      out_specs=pl.BlockSpec((_RB, D), lambda i: (i, 0)),
      out_shape=jax.ShapeDtypeStruct((N, D), jnp.float32),
  )(q, degp, W, b2)


def kernel(x, edge_index, W, b):
  src = edge_index[0]
  dst = edge_index[1]
  loop = jnp.arange(N, dtype=jnp.int32)
  pad = EPAD - (E + N)
  srcp = jnp.concatenate([src, loop, jnp.zeros((pad,), jnp.int32)])
  dstp = jnp.concatenate([dst, loop, jnp.full((pad,), N, jnp.int32)])

  zeros128 = jnp.zeros((BATCH, D), jnp.float32)
  zeros16 = jnp.zeros((BATCH, DEGW), jnp.float32)
  ones16 = jnp.ones((BATCH, DEGW), jnp.float32)

  degp = _sc_degree(dstp, ones16, zeros16)
  g0 = _tc_scale(x, degp)
  p = _sc_propagate(g0, srcp, dstp, zeros128)
  g1 = _tc_combine(p, degp)
  q = _sc_propagate(g1, srcp, dstp, zeros128)
  return _tc_final(q, degp, W, b.reshape(1, D))


# speedup vs baseline: 1.1759x; 1.1247x over previous
import functools

import jax
import jax.numpy as jnp
from jax import lax
from jax.experimental import pallas as pl
from jax.experimental.pallas import tpu as pltpu
from jax.experimental.pallas import tpu_sc as plsc

N = 10000
E = 320000
D = 128

NC = 2
NS = 16
NW = NC * NS

BATCH = 128
NB = 81
NPAIR = NB // 2
EPT = NB * BATCH
EPAD = NW * EPT

NA = 10240
RPT = NA // NS
DEGW = 16

_MESH = dict(core_axis_name="c", subcore_axis_name="s", num_cores=NC,
             num_subcores=NS)

_CHUNKS = [(j * BATCH, min(BATCH, RPT - j * BATCH))
           for j in range((RPT + BATCH - 1) // BATCH)]


def _sc_degree(dstp, ones16, zeros16):
  @functools.partial(
      pl.kernel,
      out_type=jax.ShapeDtypeStruct((NC * NA, DEGW), jnp.float32),
      mesh=plsc.VectorSubcoreMesh(**_MESH),
      scratch_types=[
          pltpu.VMEM_SHARED((NA, DEGW), jnp.float32),
          pltpu.VMEM((BATCH,), jnp.int32),
          pltpu.VMEM((BATCH,), jnp.int32),
          pltpu.VMEM((BATCH, DEGW), jnp.float32),
          pltpu.VMEM((BATCH, DEGW), jnp.float32),
          pltpu.SemaphoreType.DMA,
          pltpu.SemaphoreType.DMA,
      ],
  )
  def k(dst_hbm, ones_hbm, z_hbm, out_hbm, acc, didx0, didx1, ones_v, z_v,
        sem0, sem1):
    cid = lax.axis_index("c")
    sid = lax.axis_index("s")
    base = (cid * NS + sid) * EPT
    pltpu.sync_copy(ones_hbm, ones_v)
    pltpu.sync_copy(z_hbm, z_v)
    for o, n in _CHUNKS:
      pltpu.sync_copy(z_v.at[pl.ds(0, n)], acc.at[pl.ds(sid * RPT + o, n)])
    plsc.subcore_barrier()

    def issue(t_off, didx, sem):
      pltpu.async_copy(dst_hbm.at[pl.ds(t_off, BATCH)], didx, sem)

    def proc(t_off, didx, sem):
      pltpu.make_async_copy(dst_hbm.at[pl.ds(t_off, BATCH)], didx, sem).wait()
      pltpu.sync_copy(ones_v, acc.at[didx], add=True)

    issue(base, didx0, sem0)

    def pair(i, carry):
      o = base + 2 * i * BATCH
      issue(o + BATCH, didx1, sem1)
      proc(o, didx0, sem0)
      issue(o + 2 * BATCH, didx0, sem0)
      proc(o + BATCH, didx1, sem1)
      return carry

    lax.fori_loop(0, NPAIR, pair, 0)
    proc(base + (NB - 1) * BATCH, didx0, sem0)
    plsc.subcore_barrier()
    for o, n in _CHUNKS:
      r0 = sid * RPT + o
      pltpu.sync_copy(acc.at[pl.ds(r0, n)],
                      out_hbm.at[pl.ds(cid * NA + r0, n)])

  return k(dstp, ones16, zeros16).reshape(NC, NA, DEGW)


def _sc_propagate(g, srcp, dstp, zeros128):
  @functools.partial(
      pl.kernel,
      out_type=jax.ShapeDtypeStruct((NC * NA, D), jnp.float32),
      mesh=plsc.VectorSubcoreMesh(**_MESH),
      scratch_types=[
          pltpu.VMEM_SHARED((NA, D), jnp.float32),
          pltpu.VMEM((EPT,), jnp.int32),
          pltpu.VMEM((BATCH,), jnp.int32),
          pltpu.VMEM((BATCH,), jnp.int32),
          pltpu.VMEM((BATCH, D), jnp.float32),
          pltpu.VMEM((BATCH, D), jnp.float32),
          pltpu.SemaphoreType.DMA,
          pltpu.SemaphoreType.DMA,
      ],
  )
  def k(g_hbm, src_hbm, dst_hbm, z_hbm, out_hbm, acc, sidxs,
        didx0, didx1, rows0, rows1, sem0, sem1):
    cid = lax.axis_index("c")
    sid = lax.axis_index("s")
    wid = cid * NS + sid
    base = wid * EPT
    pltpu.sync_copy(src_hbm.at[pl.ds(base, EPT)], sidxs)
    pltpu.sync_copy(z_hbm, rows0)
    for o, n in _CHUNKS:
      pltpu.sync_copy(rows0.at[pl.ds(0, n)], acc.at[pl.ds(sid * RPT + o, n)])
    plsc.subcore_barrier()

    def issue(t, rows, sem):
      pltpu.async_copy(g_hbm.at[sidxs.at[pl.ds(t * BATCH, BATCH)]], rows, sem)

    def proc(t, didx, rows, sem):
      pltpu.sync_copy(dst_hbm.at[pl.ds(base + t * BATCH, BATCH)], didx)
      pltpu.make_async_copy(
          g_hbm.at[sidxs.at[pl.ds(t * BATCH, BATCH)]], rows, sem).wait()
      pltpu.sync_copy(rows, acc.at[didx], add=True)

    issue(0, rows0, sem0)

    def pair(i, carry):
      t = 2 * i
      issue(t + 1, rows1, sem1)
      proc(t, didx0, rows0, sem0)
      issue(t + 2, rows0, sem0)
      proc(t + 1, didx1, rows1, sem1)
      return carry

    lax.fori_loop(0, NPAIR, pair, 0)
    proc(NB - 1, didx0, rows0, sem0)
    plsc.subcore_barrier()
    for o, n in _CHUNKS:
      r0 = sid * RPT + o
      pltpu.sync_copy(acc.at[pl.ds(r0, n)],
                      out_hbm.at[pl.ds(cid * NA + r0, n)])

  return k(g, srcp, dstp, zeros128).reshape(NC, NA, D)


_RB = 400


def _deg_col(dref):
  deg = dref[0] + dref[1]
  return deg[:, 0:1]


def _tc_scale(x, degp):
  def body(x_ref, d_ref, o_ref):
    o_ref[...] = x_ref[...] * lax.rsqrt(_deg_col(d_ref))

  return pl.pallas_call(
      body,
      grid=(N // _RB,),
      in_specs=[
          pl.BlockSpec((_RB, D), lambda i: (i, 0)),
          pl.BlockSpec((NC, _RB, DEGW), lambda i: (0, i, 0)),
      ],
      out_specs=pl.BlockSpec((_RB, D), lambda i: (i, 0)),
      out_shape=jax.ShapeDtypeStruct((N, D), jnp.float32),
  )(x, degp)


def _tc_combine(p, degp):
  def body(p_ref, d_ref, o_ref):
    o_ref[...] = (p_ref[0] + p_ref[1]) / _deg_col(d_ref)

  return pl.pallas_call(
      body,
      grid=(N // _RB,),
      in_specs=[
          pl.BlockSpec((NC, _RB, D), lambda i: (0, i, 0)),
          pl.BlockSpec((NC, _RB, DEGW), lambda i: (0, i, 0)),
      ],
      out_specs=pl.BlockSpec((_RB, D), lambda i: (i, 0)),
      out_shape=jax.ShapeDtypeStruct((N, D), jnp.float32),
  )(p, degp)


def _tc_final(q, degp, W, b2):
  def body(q_ref, d_ref, w_ref, b_ref, o_ref):
    h2 = (q_ref[0] + q_ref[1]) * lax.rsqrt(_deg_col(d_ref))
    o_ref[...] = lax.dot_general(
        h2, w_ref[...], (((1,), (1,)), ((), ())),
        preferred_element_type=jnp.float32) + b_ref[...]

  return pl.pallas_call(
      body,
      grid=(N // _RB,),
      in_specs=[
          pl.BlockSpec((NC, _RB, D), lambda i: (0, i, 0)),
          pl.BlockSpec((NC, _RB, DEGW), lambda i: (0, i, 0)),
          pl.BlockSpec((D, D), lambda i: (0, 0)),
          pl.BlockSpec((1, D), lambda i: (0, 0)),
      ],
      out_specs=pl.BlockSpec((_RB, D), lambda i: (i, 0)),
      out_shape=jax.ShapeDtypeStruct((N, D), jnp.float32),
  )(q, degp, W, b2)


def kernel(x, edge_index, W, b):
  src = edge_index[0]
  dst = edge_index[1]
  loop = jnp.arange(N, dtype=jnp.int32)
  pad = EPAD - (E + N)
  srcp = jnp.concatenate([src, loop, jnp.zeros((pad,), jnp.int32)])
  dstp = jnp.concatenate([dst, loop, jnp.full((pad,), N, jnp.int32)])

  zeros128 = jnp.zeros((BATCH, D), jnp.float32)
  zeros16 = jnp.zeros((BATCH, DEGW), jnp.float32)
  ones16 = jnp.ones((BATCH, DEGW), jnp.float32)

  degp = _sc_degree(dstp, ones16, zeros16)
  g0 = _tc_scale(x, degp)
  p = _sc_propagate(g0, srcp, dstp, zeros128)
  g1 = _tc_combine(p, degp)
  q = _sc_propagate(g1, srcp, dstp, zeros128)
  return _tc_final(q, degp, W, b.reshape(1, D))


# speedup vs baseline: 1.2002x; 1.0207x over previous
import functools

import jax
import jax.numpy as jnp
from jax import lax
from jax.experimental import pallas as pl
from jax.experimental.pallas import tpu as pltpu
from jax.experimental.pallas import tpu_sc as plsc

N = 10000
E = 320000
D = 128

NC = 2
NS = 16
NW = NC * NS

BATCH = 128
NB = 81
NPAIR = NB // 2
EPT = NB * BATCH
EPAD = NW * EPT

NA = 10240
RPT = NA // NS
DEGW = 16

_MESH = dict(core_axis_name="c", subcore_axis_name="s", num_cores=NC,
             num_subcores=NS)

_CHUNKS = [(j * BATCH, min(BATCH, RPT - j * BATCH))
           for j in range((RPT + BATCH - 1) // BATCH)]


def _sc_degree(dstp, ones16, zeros16):
  @functools.partial(
      pl.kernel,
      out_type=jax.ShapeDtypeStruct((NC * NA, DEGW), jnp.float32),
      mesh=plsc.VectorSubcoreMesh(**_MESH),
      scratch_types=[
          pltpu.VMEM_SHARED((NA, DEGW), jnp.float32),
          pltpu.VMEM((BATCH,), jnp.int32),
          pltpu.VMEM((BATCH,), jnp.int32),
          pltpu.VMEM((BATCH, DEGW), jnp.float32),
          pltpu.VMEM((BATCH, DEGW), jnp.float32),
          pltpu.SemaphoreType.DMA,
          pltpu.SemaphoreType.DMA,
      ],
  )
  def k(dst_hbm, ones_hbm, z_hbm, out_hbm, acc, didx0, didx1, ones_v, z_v,
        sem0, sem1):
    cid = lax.axis_index("c")
    sid = lax.axis_index("s")
    base = (cid * NS + sid) * EPT
    pltpu.sync_copy(ones_hbm, ones_v)
    pltpu.sync_copy(z_hbm, z_v)
    for o, n in _CHUNKS:
      pltpu.sync_copy(z_v.at[pl.ds(0, n)], acc.at[pl.ds(sid * RPT + o, n)])
    plsc.subcore_barrier()

    def issue(t_off, didx, sem):
      pltpu.async_copy(dst_hbm.at[pl.ds(t_off, BATCH)], didx, sem)

    def proc(t_off, didx, sem):
      pltpu.make_async_copy(dst_hbm.at[pl.ds(t_off, BATCH)], didx, sem).wait()
      pltpu.sync_copy(ones_v, acc.at[didx], add=True)

    issue(base, didx0, sem0)

    def pair(i, carry):
      o = base + 2 * i * BATCH
      issue(o + BATCH, didx1, sem1)
      proc(o, didx0, sem0)
      issue(o + 2 * BATCH, didx0, sem0)
      proc(o + BATCH, didx1, sem1)
      return carry

    lax.fori_loop(0, NPAIR, pair, 0)
    proc(base + (NB - 1) * BATCH, didx0, sem0)
    plsc.subcore_barrier()
    for o, n in _CHUNKS:
      r0 = sid * RPT + o
      pltpu.sync_copy(acc.at[pl.ds(r0, n)],
                      out_hbm.at[pl.ds(cid * NA + r0, n)])

  return k(dstp, ones16, zeros16).reshape(NC, NA, DEGW)


def _sc_propagate(g, srcp, dstp, zeros128):
  @functools.partial(
      pl.kernel,
      out_type=jax.ShapeDtypeStruct((NC * NA, D), jnp.float32),
      mesh=plsc.VectorSubcoreMesh(**_MESH),
      scratch_types=[
          pltpu.VMEM_SHARED((NA, D), jnp.float32),
          pltpu.VMEM((EPT,), jnp.int32),
          pltpu.VMEM((BATCH,), jnp.int32),
          pltpu.VMEM((BATCH,), jnp.int32),
          pltpu.VMEM((BATCH, D), jnp.float32),
          pltpu.VMEM((BATCH, D), jnp.float32),
          pltpu.SemaphoreType.DMA,
          pltpu.SemaphoreType.DMA,
          pltpu.SemaphoreType.DMA,
          pltpu.SemaphoreType.DMA,
      ],
  )
  def k(g_hbm, src_hbm, dst_hbm, z_hbm, out_hbm, acc, sidxs,
        didx0, didx1, rows0, rows1, sem0, sem1, dsem0, dsem1):
    cid = lax.axis_index("c")
    sid = lax.axis_index("s")
    wid = cid * NS + sid
    base = wid * EPT
    pltpu.sync_copy(src_hbm.at[pl.ds(base, EPT)], sidxs)
    pltpu.sync_copy(z_hbm, rows0)
    for o, n in _CHUNKS:
      pltpu.sync_copy(rows0.at[pl.ds(0, n)], acc.at[pl.ds(sid * RPT + o, n)])
    plsc.subcore_barrier()

    def issue(t, didx, rows, sem, dsem):
      pltpu.async_copy(g_hbm.at[sidxs.at[pl.ds(t * BATCH, BATCH)]], rows, sem)
      pltpu.async_copy(dst_hbm.at[pl.ds(base + t * BATCH, BATCH)], didx, dsem)

    def proc(t, didx, rows, sem, dsem):
      pltpu.make_async_copy(
          dst_hbm.at[pl.ds(base + t * BATCH, BATCH)], didx, dsem).wait()
      pltpu.make_async_copy(
          g_hbm.at[sidxs.at[pl.ds(t * BATCH, BATCH)]], rows, sem).wait()
      pltpu.sync_copy(rows, acc.at[didx], add=True)

    issue(0, didx0, rows0, sem0, dsem0)

    def pair(i, carry):
      t = 2 * i
      issue(t + 1, didx1, rows1, sem1, dsem1)
      proc(t, didx0, rows0, sem0, dsem0)
      issue(t + 2, didx0, rows0, sem0, dsem0)
      proc(t + 1, didx1, rows1, sem1, dsem1)
      return carry

    lax.fori_loop(0, NPAIR, pair, 0)
    proc(NB - 1, didx0, rows0, sem0, dsem0)
    plsc.subcore_barrier()
    for o, n in _CHUNKS:
      r0 = sid * RPT + o
      pltpu.sync_copy(acc.at[pl.ds(r0, n)],
                      out_hbm.at[pl.ds(cid * NA + r0, n)])

  return k(g, srcp, dstp, zeros128).reshape(NC, NA, D)


_RB = 400


def _deg_col(dref):
  deg = dref[0] + dref[1]
  return deg[:, 0:1]


def _tc_scale(x, degp):
  def body(x_ref, d_ref, o_ref):
    o_ref[...] = x_ref[...] * lax.rsqrt(_deg_col(d_ref))

  return pl.pallas_call(
      body,
      grid=(N // _RB,),
      in_specs=[
          pl.BlockSpec((_RB, D), lambda i: (i, 0)),
          pl.BlockSpec((NC, _RB, DEGW), lambda i: (0, i, 0)),
      ],
      out_specs=pl.BlockSpec((_RB, D), lambda i: (i, 0)),
      out_shape=jax.ShapeDtypeStruct((N, D), jnp.float32),
  )(x, degp)


def _tc_combine(p, degp):
  def body(p_ref, d_ref, o_ref):
    o_ref[...] = (p_ref[0] + p_ref[1]) / _deg_col(d_ref)

  return pl.pallas_call(
      body,
      grid=(N // _RB,),
      in_specs=[
          pl.BlockSpec((NC, _RB, D), lambda i: (0, i, 0)),
          pl.BlockSpec((NC, _RB, DEGW), lambda i: (0, i, 0)),
      ],
      out_specs=pl.BlockSpec((_RB, D), lambda i: (i, 0)),
      out_shape=jax.ShapeDtypeStruct((N, D), jnp.float32),
  )(p, degp)


def _tc_final(q, degp, W, b2):
  def body(q_ref, d_ref, w_ref, b_ref, o_ref):
    h2 = (q_ref[0] + q_ref[1]) * lax.rsqrt(_deg_col(d_ref))
    o_ref[...] = lax.dot_general(
        h2, w_ref[...], (((1,), (1,)), ((), ())),
        preferred_element_type=jnp.float32) + b_ref[...]

  return pl.pallas_call(
      body,
      grid=(N // _RB,),
      in_specs=[
          pl.BlockSpec((NC, _RB, D), lambda i: (0, i, 0)),
          pl.BlockSpec((NC, _RB, DEGW), lambda i: (0, i, 0)),
          pl.BlockSpec((D, D), lambda i: (0, 0)),
          pl.BlockSpec((1, D), lambda i: (0, 0)),
      ],
      out_specs=pl.BlockSpec((_RB, D), lambda i: (i, 0)),
      out_shape=jax.ShapeDtypeStruct((N, D), jnp.float32),
  )(q, degp, W, b2)


def kernel(x, edge_index, W, b):
  src = edge_index[0]
  dst = edge_index[1]
  loop = jnp.arange(N, dtype=jnp.int32)
  pad = EPAD - (E + N)
  srcp = jnp.concatenate([src, loop, jnp.zeros((pad,), jnp.int32)])
  dstp = jnp.concatenate([dst, loop, jnp.full((pad,), N, jnp.int32)])

  zeros128 = jnp.zeros((BATCH, D), jnp.float32)
  zeros16 = jnp.zeros((BATCH, DEGW), jnp.float32)
  ones16 = jnp.ones((BATCH, DEGW), jnp.float32)

  degp = _sc_degree(dstp, ones16, zeros16)
  g0 = _tc_scale(x, degp)
  p = _sc_propagate(g0, srcp, dstp, zeros128)
  g1 = _tc_combine(p, degp)
  q = _sc_propagate(g1, srcp, dstp, zeros128)
  return _tc_final(q, degp, W, b.reshape(1, D))
